# baseline (speedup 1.0000x reference)
import functools

import jax
import jax.numpy as jnp
from jax import lax
from jax.experimental import pallas as pl
from jax.experimental.pallas import tpu as pltpu
from jax.experimental.pallas import tpu_sc as plsc

R = 128
C = 32768
NCORE = 2
NSUB = 16
L = 16
NW = NCORE * NSUB
RPW = R // NW
NACC = 8
VPB = L * NACC
NIT = C // VPB
BIG = 0x7FFFFFFF

_scratch_types = [
    pltpu.VMEM((C,), jnp.float32),
    pltpu.VMEM((C,), jnp.float32),
    pltpu.VMEM((L,), jnp.int32),
    pltpu.SemaphoreType.DMA,
    pltpu.SemaphoreType.DMA,
]


def _argmax_body(x_hbm, out_hbm, buf0, buf1, res_v, sem0, sem1):
    wid = lax.axis_index("s") * NCORE + lax.axis_index("c")
    row0 = wid * RPW
    bufs = (buf0, buf1)
    sems = (sem0, sem1)
    lanes = lax.iota(jnp.int32, L)

    pltpu.make_async_copy(x_hbm.at[pl.ds(row0 * C, C)], buf0, sem0).start()
    pltpu.make_async_copy(x_hbm.at[pl.ds((row0 + 1) * C, C)], buf1, sem1).start()

    resvec = jnp.zeros((L,), jnp.int32)
    for j in range(RPW):
        buf = bufs[j % 2]
        sem = sems[j % 2]
        pltpu.make_async_copy(
            x_hbm.at[pl.ds((row0 + j) * C, C)], buf, sem
        ).wait()

        neg = jnp.full((L,), -jnp.inf, jnp.float32)
        zer = jnp.zeros((L,), jnp.int32)
        init = (tuple(neg for _ in range(NACC)),
                tuple(zer for _ in range(NACC)))

        def body(it, carry, buf=buf):
            best, bidx = carry
            base = it * VPB
            nb = []
            ni = []
            for a in range(NACC):
                off = base + a * L
                v = buf[pl.ds(off, L)]
                idx = jnp.full((L,), off, jnp.int32) + lanes
                m = v > best[a]
                nb.append(jnp.where(m, v, best[a]))
                ni.append(jnp.where(m, idx, bidx[a]))
            return tuple(nb), tuple(ni)

        best, bidx = lax.fori_loop(0, NIT, body, init)

        if j + 2 < RPW:
            pltpu.make_async_copy(
                x_hbm.at[pl.ds((row0 + j + 2) * C, C)], buf, sem
            ).start()

        cb, ci = best[0], bidx[0]
        for a in range(1, NACC):
            take = (best[a] > cb) | ((best[a] == cb) & (bidx[a] < ci))
            cb = jnp.where(take, best[a], cb)
            ci = jnp.where(take, bidx[a], ci)

        rowmax = cb
        for sh in (8, 4, 2, 1):
            rot = (lanes + sh) & (L - 1)
            rowmax = jnp.maximum(
                rowmax, rowmax.at[rot].get(mode="promise_in_bounds")
            )
        cand = jnp.where(cb == rowmax, ci, jnp.full((L,), BIG, jnp.int32))
        for sh in (8, 4, 2, 1):
            rot = (lanes + sh) & (L - 1)
            cand = jnp.minimum(
                cand, cand.at[rot].get(mode="promise_in_bounds")
            )
        resvec = jnp.where(lanes == j, cand, resvec)

    res_v[...] = resvec
    pltpu.sync_copy(res_v, out_hbm.at[pl.ds(wid * L, L)])


@functools.cache
def _get_argmax_sc():
    mesh = plsc.VectorSubcoreMesh(
        core_axis_name="c",
        subcore_axis_name="s",
        num_cores=NCORE,
        num_subcores=NSUB,
    )
    return pl.kernel(
        _argmax_body,
        out_type=jax.ShapeDtypeStruct((NW * L,), jnp.int32),
        mesh=mesh,
        scratch_types=_scratch_types,
    )


def kernel(x):
    out = _get_argmax_sc()(x.reshape(R * C))
    out = out.reshape(NW, L)[:, :RPW].reshape(R)
    return out.astype(jnp.int64)
